# baseline (speedup 1.0000x reference)
import functools

import jax
import jax.numpy as jnp
from jax import lax
from jax.experimental import pallas as pl
from jax.experimental.pallas import tpu as pltpu
from jax.experimental.pallas import tpu_sc as plsc

_BATCH = 16384
_N_FIELDS = 26
_NC = 2
_NS = 16
_NW = _NC * _NS
_B_PER_W = _BATCH // _NW
_IDX_PER_W = _B_PER_W * _N_FIELDS
_LANES = 16


def _fm_linear_body(x_hbm, w_hbm, b_hbm, out_hbm, idx_v, rows_v, out_v,
                    bias_v, sem, gsem):
    wid = lax.axis_index("s") * _NC + lax.axis_index("c")
    base = wid * _B_PER_W

    cp = pltpu.async_copy(
        x_hbm.at[pl.ds(wid * _IDX_PER_W, _IDX_PER_W)], idx_v, sem)
    pltpu.sync_copy(b_hbm, bias_v)
    cp.wait()

    pltpu.async_copy(w_hbm.at[idx_v], rows_v, gsem).wait()

    bias_vec = bias_v[...]
    row_off = lax.iota(jnp.int32, _LANES) * _N_FIELDS

    def step(blk, carry):
        acc = bias_vec
        idx0 = row_off + blk * (_LANES * _N_FIELDS)
        for f in range(_N_FIELDS):
            acc = acc + plsc.load_gather(rows_v, [idx0 + f])
        out_v[pl.ds(blk * _LANES, _LANES)] = acc
        return carry

    lax.fori_loop(0, _B_PER_W // _LANES, step, 0)

    pltpu.sync_copy(out_v, out_hbm.at[pl.ds(base, _B_PER_W)])


_fm_linear = functools.partial(
    pl.kernel,
    mesh=plsc.VectorSubcoreMesh(core_axis_name="c", subcore_axis_name="s"),
    out_type=jax.ShapeDtypeStruct((_BATCH,), jnp.float32),
    scratch_types=[
        pltpu.VMEM((_IDX_PER_W,), jnp.int32),
        pltpu.VMEM((_IDX_PER_W,), jnp.float32),
        pltpu.VMEM((_B_PER_W,), jnp.float32),
        pltpu.VMEM((_LANES,), jnp.float32),
        pltpu.SemaphoreType.DMA,
        pltpu.SemaphoreType.DMA,
    ],
    compiler_params=pltpu.CompilerParams(needs_layout_passes=False),
)(_fm_linear_body)


def kernel(x, fc_weight, bias):
    x_flat = x.reshape(-1).astype(jnp.int32)
    w_flat = jnp.swapaxes(fc_weight, 0, 1).reshape(-1)
    bias16 = jnp.broadcast_to(bias.astype(jnp.float32), (_LANES,))
    out = _fm_linear(x_flat, w_flat, bias16)
    return out.reshape(_BATCH, 1)


# speedup vs baseline: 1.6122x; 1.6122x over previous
import functools

import jax
import jax.numpy as jnp
from jax import lax
from jax.experimental import pallas as pl
from jax.experimental.pallas import tpu as pltpu
from jax.experimental.pallas import tpu_sc as plsc

_BATCH = 16384
_N_FIELDS = 26
_NC = 2
_NS = 16
_NW = _NC * _NS
_B_PER_W = _BATCH // _NW
_IDX_PER_W = _B_PER_W * _N_FIELDS
_LANES = 16


def _fm_linear_body(x_hbm, w_hbm, b_hbm, out_hbm, idx_v, rows_v, out_v,
                    bias_v, sem, gsem):
    wid = lax.axis_index("s") * _NC + lax.axis_index("c")
    base = wid * _B_PER_W

    cp = pltpu.async_copy(
        x_hbm.at[pl.ds(wid * _IDX_PER_W, _IDX_PER_W)], idx_v, sem)
    pltpu.sync_copy(b_hbm, bias_v)
    cp.wait()

    pltpu.async_copy(w_hbm.at[idx_v], rows_v, gsem).wait()

    bias_vec = bias_v[...]
    row_off = lax.iota(jnp.int32, _LANES) * _N_FIELDS

    def step(blk, carry):
        acc = bias_vec
        idx0 = row_off + blk * (_LANES * _N_FIELDS)
        for f in range(_N_FIELDS):
            acc = acc + plsc.load_gather(rows_v, [idx0 + f])
        out_v[pl.ds(blk * _LANES, _LANES)] = acc
        return carry

    lax.fori_loop(0, _B_PER_W // _LANES, step, 0)

    pltpu.sync_copy(out_v, out_hbm.at[pl.ds(base, _B_PER_W)])


_fm_linear = functools.partial(
    pl.kernel,
    mesh=plsc.VectorSubcoreMesh(core_axis_name="c", subcore_axis_name="s"),
    out_type=jax.ShapeDtypeStruct((_BATCH,), jnp.float32),
    scratch_types=[
        pltpu.VMEM((_IDX_PER_W,), jnp.int32),
        pltpu.VMEM((_IDX_PER_W,), jnp.float32),
        pltpu.VMEM((_B_PER_W,), jnp.float32),
        pltpu.VMEM((_LANES,), jnp.float32),
        pltpu.SemaphoreType.DMA,
        pltpu.SemaphoreType.DMA,
    ],
    compiler_params=pltpu.CompilerParams(needs_layout_passes=False),
)(_fm_linear_body)


def kernel(x, fc_weight, bias):
    x_flat = x.reshape(-1).astype(jnp.int32)
    w_flat = jnp.pad(fc_weight, ((0, 384), (0, 0))).reshape(-1)
    bias16 = jnp.broadcast_to(bias.astype(jnp.float32), (_LANES,))
    out = _fm_linear(x_flat, w_flat, bias16)
    return out.reshape(_BATCH, 1)
